# Initial kernel scaffold; baseline (speedup 1.0000x reference)
import jax
import jax.numpy as jnp
from jax.experimental import pallas as pl


def kernel(x, W_lin, bias, W_cross):
    raise NotImplementedError("write your pallas kernel here")


# speedup vs baseline: 1.3167x; 1.3167x over previous
import jax
import jax.numpy as jnp
from jax import lax
from jax.experimental import pallas as pl
from jax.experimental.pallas import tpu as pltpu
from jax.experimental.pallas import tpu_sc as plsc

B = 16384
F = 26
D = 16
TOT = 26 * 40000
NC = 2
NS = 16
NW = NC * NS
BPW = B // NW
CH = 64
NCHUNK = BPW // CH
ROWS_PER_CHUNK = CH * F
GPC = ROWS_PER_CHUNK // 128
IROWS = (F * BPW) // 128


def _fm_body(idx_em, wlin, wcross, bias16, out,
             idx_em_v, rows_a, rows_b, wl_a, wl_b, z_v, rbuf, bias_v,
             sem_x0, sem_x1):
    wid = lax.axis_index("s") * NC + lax.axis_index("c")
    base = wid * BPW

    pltpu.sync_copy(idx_em.at[wid], idx_em_v)
    pltpu.sync_copy(bias16, bias_v)

    bufs = (rows_a, rows_b)
    wls = (wl_a, wl_b)
    sems = (sem_x0, sem_x1)

    def fire_chunk(c):
        buf = bufs[c % 2]
        wl = wls[c % 2]
        sem = sems[c % 2]
        cps = []
        for j in range(GPC):
            irow = idx_em_v.at[c * GPC + j]
            cps.append(pltpu.async_copy(wcross.at[irow],
                                        buf.at[pl.ds(j * 128, 128)], sem))
            cps.append(pltpu.async_copy(wlin.at[irow],
                                        wl.at[pl.ds(j * 128, 128)], sem))
        return cps

    pend = fire_chunk(0)
    for c in range(NCHUNK):
        if c + 1 < NCHUNK:
            nxt = fire_chunk(c + 1)
        for cp in pend:
            cp.wait()
        pend = nxt if c + 1 < NCHUNK else []
        buf = bufs[c % 2]
        wl = wls[c % 2]

        def elem_body(e, carry, buf=buf):
            b0 = e * F
            r0 = buf[b0]
            r1 = buf[b0 + 1]
            s0, s1 = r0, r1
            q0, q1 = r0 * r0, r1 * r1
            for f in range(2, F, 2):
                ra = buf[b0 + f]
                rb = buf[b0 + f + 1]
                s0 = s0 + ra
                s1 = s1 + rb
                q0 = q0 + ra * ra
                q1 = q1 + rb * rb
            s = s0 + s1
            q = q0 + q1
            rbuf[e, pl.ds(0, D)] = s * s - q
            return carry

        lax.fori_loop(0, CH, elem_body, 0)

        lanes = jnp.arange(D, dtype=jnp.int32)
        for g in range(CH // D):
            rid = lanes + (g * D)
            acc = plsc.load_gather(rbuf, [rid, jnp.zeros((D,), jnp.int32)])
            for d in range(1, D):
                acc = acc + plsc.load_gather(
                    rbuf, [rid, jnp.full((D,), d, jnp.int32)])
            ridf = rid * F
            la = plsc.load_gather(wl, [ridf])
            lb = plsc.load_gather(wl, [ridf + 1])
            for f in range(2, F, 2):
                la = la + plsc.load_gather(wl, [ridf + f])
                lb = lb + plsc.load_gather(wl, [ridf + (f + 1)])
            z_v[pl.ds(c * CH + g * D, D)] = 0.5 * acc + (la + lb)

    bvec = bias_v[...]
    for i in range(BPW // D):
        v = z_v[pl.ds(i * D, D)] + bvec
        z_v[pl.ds(i * D, D)] = 1.0 / (1.0 + jnp.exp(-v))

    pltpu.sync_copy(z_v, out.at[pl.ds(base, BPW)])


@jax.jit
def kernel(x, W_lin, bias, W_cross):
    offs = jnp.arange(F, dtype=jnp.int32) * 40000
    idx = x.astype(jnp.int32) + offs[None, :]
    idx_em = idx.reshape(NW, IROWS, 128)
    wlin = W_lin.reshape(TOT)
    bias16 = jnp.broadcast_to(bias.astype(jnp.float32), (D,))

    mesh = plsc.VectorSubcoreMesh(core_axis_name="c", subcore_axis_name="s")
    fm = pl.kernel(
        _fm_body,
        out_type=jax.ShapeDtypeStruct((B,), jnp.float32),
        mesh=mesh,
        scratch_types=[
            pltpu.VMEM((IROWS, 128), jnp.int32),
            pltpu.VMEM((ROWS_PER_CHUNK, D), jnp.float32),
            pltpu.VMEM((ROWS_PER_CHUNK, D), jnp.float32),
            pltpu.VMEM((ROWS_PER_CHUNK,), jnp.float32),
            pltpu.VMEM((ROWS_PER_CHUNK,), jnp.float32),
            pltpu.VMEM((BPW,), jnp.float32),
            pltpu.VMEM((CH, D + 1), jnp.float32),
            pltpu.VMEM((D,), jnp.float32),
            pltpu.SemaphoreType.DMA,
            pltpu.SemaphoreType.DMA,
        ],
        compiler_params=pltpu.CompilerParams(
            needs_layout_passes=False, use_tc_tiling_on_sc=False),
    )
    out = fm(idx_em, wlin, W_cross, bias16)
    return out.reshape(B, 1)
